# baseline (speedup 1.0000x reference)
import functools

import jax
import jax.numpy as jnp
from jax import lax
from jax.experimental import pallas as pl
from jax.experimental.pallas import tpu as pltpu
from jax.experimental.pallas import tpu_sc as plsc

NUM_EXPERTS = 8
TOP_K = 2
DIM = 768
T = 2048
BLK = 256
NBLK = TOP_K * T // BLK + NUM_EXPERTS
PMAX = NBLK * BLK
CH = 256
NW = 32
TPW = T // NW
HALF = TPW // 2


def _router_body(x_ref, wr_ref, br_ref, be_ref,
                 x2_ref, dest_ref, cnt_ref, offs_ref, bias_ref, aux_ref,
                 csum_ref):
    iota = lax.broadcasted_iota(jnp.int32, (CH, NUM_EXPERTS), 1)
    lane = lax.broadcasted_iota(jnp.int32, (CH, 128), 1)
    tril = (lax.broadcasted_iota(jnp.int32, (CH, CH), 0) >=
            lax.broadcasted_iota(jnp.int32, (CH, CH), 1)).astype(jnp.float32)

    run = jnp.zeros((1, NUM_EXPERTS), jnp.float32)
    fsum = jnp.zeros((1, NUM_EXPERTS), jnp.float32)
    psum = jnp.zeros((1, NUM_EXPERTS), jnp.float32)
    for i in range(T // CH):
        sl = pl.ds(i * CH, CH)
        xs = x_ref[sl, :]
        logits = jnp.dot(xs, wr_ref[...],
                         preferred_element_type=jnp.float32) + br_ref[...]
        mx = jnp.max(logits, axis=1, keepdims=True)
        ex = jnp.exp(logits - mx)
        probs = ex / jnp.sum(ex, axis=1, keepdims=True)

        m1 = jnp.max(probs, axis=1, keepdims=True)
        a1 = jnp.min(jnp.where(probs == m1, iota, NUM_EXPERTS), axis=1,
                     keepdims=True)
        sel1 = iota == a1
        rest = jnp.where(sel1, -1.0, probs)
        m2 = jnp.max(rest, axis=1, keepdims=True)
        a2 = jnp.min(jnp.where(rest == m2, iota, NUM_EXPERTS), axis=1,
                     keepdims=True)
        sel2 = iota == a2

        gate = jnp.where(sel1, m1, 0.0) + jnp.where(sel2, m2, 0.0)
        bias_ref[sl, :] = jnp.dot(gate, be_ref[...],
                                  preferred_element_type=jnp.float32)
        x2_ref[0, sl, :] = m1 * xs
        x2_ref[1, sl, :] = m2 * xs

        assign = sel1.astype(jnp.float32) + sel2.astype(jnp.float32)
        csum = jnp.dot(tril, assign,
                       preferred_element_type=jnp.float32) + run
        csum_ref[sl, :] = csum
        run = csum[CH - 1:CH, :]

        dest_ref[sl, :] = jnp.where(lane == 0, a1,
                                    jnp.where(lane == 1, a2, 0))

        fsum = fsum + jnp.sum(assign, axis=0, keepdims=True)
        psum = psum + jnp.sum(probs, axis=0, keepdims=True)

    counts = run
    padded = jnp.floor((counts + (BLK - 1)) / BLK) * BLK
    mstrict = (lax.broadcasted_iota(jnp.int32, (NUM_EXPERTS, NUM_EXPERTS), 0)
               < lax.broadcasted_iota(jnp.int32,
                                      (NUM_EXPERTS, NUM_EXPERTS), 1)
               ).astype(jnp.float32)
    offs = jnp.dot(padded, mstrict, preferred_element_type=jnp.float32)
    cnt_ref[...] = counts.astype(jnp.int32)
    offs_ref[...] = offs.astype(jnp.int32)

    f = fsum / (T * TOP_K)
    P = psum / T
    aux_ref[0, 0] = NUM_EXPERTS * jnp.sum(f * P)

    for i in range(T // CH):
        sl = pl.ds(i * CH, CH)
        a1 = dest_ref[sl, 0:1]
        a2 = dest_ref[sl, 1:2]
        sel1 = iota == a1
        sel2 = iota == a2
        base = offs + csum_ref[sl, :] - 1.0
        d0 = jnp.sum(jnp.where(sel1, base, 0.0), axis=1, keepdims=True)
        d1 = jnp.sum(jnp.where(sel2, base, 0.0), axis=1, keepdims=True)
        dest_ref[sl, :] = jnp.where(lane == 0, d0.astype(jnp.int32),
                                    jnp.where(lane == 1, d1.astype(jnp.int32),
                                              0))


def _dispatch_body(x2_hbm, dest_hbm, xg_hbm, idx_ref, rows_ref, sem):
    wid = lax.axis_index("s") * 2 + lax.axis_index("c")
    base = wid * TPW
    for k in range(TOP_K):
        pltpu.sync_copy(dest_hbm.at[k, pl.ds(base, TPW)], idx_ref)
        pltpu.sync_copy(x2_hbm.at[k, pl.ds(base, TPW), :], rows_ref)
        pltpu.async_copy(rows_ref, xg_hbm.at[idx_ref], sem).wait()


def _gemm_body(bexp_ref, bval_ref, xg_ref, we_ref, yg_ref):
    j = pl.program_id(0)

    @pl.when(bval_ref[j] == 1)
    def _():
        yg_ref[...] = jnp.dot(xg_ref[...], we_ref[0],
                              preferred_element_type=jnp.float32)


def _combine_body(yg_hbm, dest_hbm, bias_hbm, out_hbm,
                  idx0_ref, idx1_ref, a_ref, b_ref, c_ref, sem):
    wid = lax.axis_index("s") * 2 + lax.axis_index("c")
    base = wid * TPW
    for h in range(TPW // HALF):
        hb = base + h * HALF
        pltpu.sync_copy(dest_hbm.at[0, pl.ds(hb, HALF)], idx0_ref)
        pltpu.sync_copy(dest_hbm.at[1, pl.ds(hb, HALF)], idx1_ref)
        pltpu.async_copy(yg_hbm.at[idx0_ref], a_ref, sem).wait()
        pltpu.async_copy(yg_hbm.at[idx1_ref], b_ref, sem).wait()
        pltpu.sync_copy(bias_hbm.at[pl.ds(hb, HALF), :], c_ref)

        def _row(j, carry):
            for c in range(DIM // 16):
                cs = pl.ds(c * 16, 16)
                a_ref[j, cs] = a_ref[j, cs] + b_ref[j, cs] + c_ref[j, cs]
            return carry

        lax.fori_loop(0, HALF, _row, 0)
        pltpu.sync_copy(a_ref, out_hbm.at[pl.ds(hb, HALF), :])


@jax.jit
def kernel(X, Wr, br, We, be):
    Xf = X.reshape(T, DIM)
    br2 = br.reshape(1, NUM_EXPERTS)

    x2, dest01, cnt, offs, bias, aux = pl.pallas_call(
        _router_body,
        in_specs=[
            pl.BlockSpec((T, DIM), lambda: (0, 0)),
            pl.BlockSpec((DIM, NUM_EXPERTS), lambda: (0, 0)),
            pl.BlockSpec((1, NUM_EXPERTS), lambda: (0, 0)),
            pl.BlockSpec((NUM_EXPERTS, DIM), lambda: (0, 0)),
        ],
        out_specs=[
            pl.BlockSpec((TOP_K, T, DIM), lambda: (0, 0, 0)),
            pl.BlockSpec((T, 128), lambda: (0, 0)),
            pl.BlockSpec((1, NUM_EXPERTS), lambda: (0, 0)),
            pl.BlockSpec((1, NUM_EXPERTS), lambda: (0, 0)),
            pl.BlockSpec((T, DIM), lambda: (0, 0)),
            pl.BlockSpec(memory_space=pltpu.SMEM),
        ],
        out_shape=[
            jax.ShapeDtypeStruct((TOP_K, T, DIM), jnp.float32),
            jax.ShapeDtypeStruct((T, 128), jnp.int32),
            jax.ShapeDtypeStruct((1, NUM_EXPERTS), jnp.int32),
            jax.ShapeDtypeStruct((1, NUM_EXPERTS), jnp.int32),
            jax.ShapeDtypeStruct((T, DIM), jnp.float32),
            jax.ShapeDtypeStruct((1, 1), jnp.float32),
        ],
        scratch_shapes=[pltpu.VMEM((T, NUM_EXPERTS), jnp.float32)],
    )(Xf, Wr, br2, be)

    counts = cnt[0]
    offs1 = offs[0]
    padded = ((counts + BLK - 1) // BLK) * BLK
    total = jnp.sum(padded)
    row0 = jnp.arange(NBLK, dtype=jnp.int32) * BLK
    bexp = (jnp.sum((row0[:, None] >= offs1[None, :]).astype(jnp.int32),
                    axis=1) - 1).astype(jnp.int32)
    bexp = jnp.clip(bexp, 0, NUM_EXPERTS - 1)
    bval = (row0 < total).astype(jnp.int32)
    dest2 = dest01[:, :2].T

    mesh = plsc.VectorSubcoreMesh(core_axis_name="c", subcore_axis_name="s")

    dispatch = functools.partial(
        pl.kernel,
        mesh=mesh,
        out_type=jax.ShapeDtypeStruct((PMAX, DIM), jnp.float32),
        scratch_types=[
            pltpu.VMEM((TPW,), jnp.int32),
            pltpu.VMEM((TPW, DIM), jnp.float32),
            pltpu.SemaphoreType.DMA,
        ],
    )(_dispatch_body)
    xg = dispatch(x2, dest2)

    yg = pl.pallas_call(
        _gemm_body,
        grid_spec=pltpu.PrefetchScalarGridSpec(
            num_scalar_prefetch=2,
            grid=(NBLK,),
            in_specs=[
                pl.BlockSpec((BLK, DIM), lambda j, be_, bv_: (j, 0)),
                pl.BlockSpec((1, DIM, DIM), lambda j, be_, bv_: (be_[j], 0, 0)),
            ],
            out_specs=pl.BlockSpec((BLK, DIM), lambda j, be_, bv_: (j, 0)),
        ),
        out_shape=jax.ShapeDtypeStruct((PMAX, DIM), jnp.float32),
    )(bexp, bval, xg, We)

    combine = functools.partial(
        pl.kernel,
        mesh=mesh,
        out_type=jax.ShapeDtypeStruct((T, DIM), jnp.float32),
        scratch_types=[
            pltpu.VMEM((HALF,), jnp.int32),
            pltpu.VMEM((HALF,), jnp.int32),
            pltpu.VMEM((HALF, DIM), jnp.float32),
            pltpu.VMEM((HALF, DIM), jnp.float32),
            pltpu.VMEM((HALF, DIM), jnp.float32),
            pltpu.SemaphoreType.DMA,
        ],
    )(_combine_body)
    out = combine(yg, dest2, bias)

    return out.reshape(X.shape), aux[0, 0]


# speedup vs baseline: 3.0264x; 3.0264x over previous
import jax
import jax.numpy as jnp
from jax import lax
from jax.experimental import pallas as pl
from jax.experimental.pallas import tpu as pltpu

NUM_EXPERTS = 8
TOP_K = 2
DIM = 768
T = 2048


def _moe_body(x_ref, wr_ref, br_ref, we_ref, be_ref, out_ref, aux_ref,
              gate_ref):
    e = pl.program_id(0)

    @pl.when(e == 0)
    def _router():
        x = x_ref[...]
        logits = jnp.dot(x, wr_ref[...],
                         preferred_element_type=jnp.float32) + br_ref[...]
        mx = jnp.max(logits, axis=1, keepdims=True)
        ex = jnp.exp(logits - mx)
        probs = ex / jnp.sum(ex, axis=1, keepdims=True)

        iota = lax.broadcasted_iota(jnp.int32, (T, NUM_EXPERTS), 1)
        m1 = jnp.max(probs, axis=1, keepdims=True)
        a1 = jnp.min(jnp.where(probs == m1, iota, NUM_EXPERTS), axis=1,
                     keepdims=True)
        sel1 = iota == a1
        probs_rest = jnp.where(sel1, -1.0, probs)
        m2 = jnp.max(probs_rest, axis=1, keepdims=True)
        a2 = jnp.min(jnp.where(probs_rest == m2, iota, NUM_EXPERTS), axis=1,
                     keepdims=True)
        sel2 = iota == a2

        gate = jnp.where(sel1, m1, 0.0) + jnp.where(sel2, m2, 0.0)
        gate_ref[...] = gate

        f = jnp.sum(sel1.astype(jnp.float32) + sel2.astype(jnp.float32),
                    axis=0) / (T * TOP_K)
        P = jnp.mean(probs, axis=0)
        aux_ref[0, 0] = NUM_EXPERTS * jnp.sum(f * P)

        out_ref[...] = jnp.dot(gate, be_ref[...],
                               preferred_element_type=jnp.float32)

    iota = lax.broadcasted_iota(jnp.int32, (T, NUM_EXPERTS), 1)
    g_e = jnp.sum(jnp.where(iota == e, gate_ref[...], 0.0), axis=1,
                  keepdims=True)
    out_ref[...] += g_e * jnp.dot(x_ref[...], we_ref[0],
                                  preferred_element_type=jnp.float32)


@jax.jit
def kernel(X, Wr, br, We, be):
    Xf = X.reshape(T, DIM)
    br2 = br.reshape(1, NUM_EXPERTS)

    out, aux = pl.pallas_call(
        _moe_body,
        grid=(NUM_EXPERTS,),
        in_specs=[
            pl.BlockSpec((T, DIM), lambda e: (0, 0)),
            pl.BlockSpec((DIM, NUM_EXPERTS), lambda e: (0, 0)),
            pl.BlockSpec((1, NUM_EXPERTS), lambda e: (0, 0)),
            pl.BlockSpec((1, DIM, DIM), lambda e: (e, 0, 0)),
            pl.BlockSpec((NUM_EXPERTS, DIM), lambda e: (0, 0)),
        ],
        out_specs=[
            pl.BlockSpec((T, DIM), lambda e: (0, 0)),
            pl.BlockSpec(memory_space=pltpu.SMEM),
        ],
        out_shape=[
            jax.ShapeDtypeStruct((T, DIM), jnp.float32),
            jax.ShapeDtypeStruct((1, 1), jnp.float32),
        ],
        scratch_shapes=[pltpu.VMEM((T, NUM_EXPERTS), jnp.float32)],
    )(Xf, Wr, br2, We, be)

    return out.reshape(X.shape), aux[0, 0]
